# baseline (speedup 1.0000x reference)
import functools

import jax
import jax.numpy as jnp
from jax import lax
from jax.experimental import pallas as pl
from jax.experimental.pallas import tpu as pltpu
from jax.experimental.pallas import tpu_sc as plsc

_VOCAB = 100000
_EMBED_DIM = 128
_BATCH = 4096
_HIST_LEN = 200

_NUM_CORES = 2
_NUM_SUBCORES = 16
_NW = _NUM_CORES * _NUM_SUBCORES
_B_TOTAL = _BATCH * _HIST_LEN
_B_PER_W = _B_TOTAL // _NW
_CHUNK = 128
_NCHUNK = _B_PER_W // _CHUNK


_CPB = 2
_BIG = _CHUNK * _CPB
_NBIG = _B_PER_W // _BIG


def _gather_body(idx_hbm, table_hbm, out_hbm, idx_v,
                 buf_a, buf_b, gs_a, gs_b, ws_a, ws_b):
    bufs = (buf_a, buf_b)
    gsem = (gs_a, gs_b)
    wsem = (ws_a, ws_b)
    wid = lax.axis_index("s") * _NUM_CORES + lax.axis_index("c")
    pltpu.sync_copy(idx_hbm.at[wid], idx_v)
    base = wid * _B_PER_W

    def start_gathers(j, k):
        for c in range(_CPB):
            pltpu.async_copy(table_hbm.at[idx_v.at[_CPB * j + c]],
                             bufs[k].at[pl.ds(c * _CHUNK, _CHUNK)], gsem[k])

    def wait_gathers(j, k):
        for c in range(_CPB):
            pltpu.make_async_copy(table_hbm.at[idx_v.at[_CPB * j + c]],
                                  bufs[k].at[pl.ds(c * _CHUNK, _CHUNK)],
                                  gsem[k]).wait()

    def out_slice(j):
        return out_hbm.at[pl.ds(base + j * _BIG, _BIG)]

    def start_write(j, k):
        pltpu.async_copy(bufs[k], out_slice(j), wsem[k])

    def wait_write(j, k):
        pltpu.make_async_copy(bufs[k], out_slice(j), wsem[k]).wait()

    def step(j, bj):
        kn = 1 - bj
        wait_write(j - 1, kn)
        start_gathers(j + 1, kn)
        wait_gathers(j, bj)
        start_write(j, bj)

    start_gathers(0, 0)
    start_gathers(1, 1)
    wait_gathers(0, 0)
    start_write(0, 0)

    def group(g, carry):
        for i in range(2):
            step(1 + 2 * g + i, (1 + i) % 2)
        return carry

    lax.fori_loop(0, (_NBIG - 2) // 2, group, 0)

    wait_write(_NBIG - 2, 0)
    wait_gathers(_NBIG - 1, 1)
    start_write(_NBIG - 1, 1)
    wait_write(_NBIG - 1, 1)


_mesh = plsc.VectorSubcoreMesh(core_axis_name="c", subcore_axis_name="s")

_gather_call = pl.kernel(
    _gather_body,
    out_type=jax.ShapeDtypeStruct((_B_TOTAL, _EMBED_DIM), jnp.float32),
    mesh=_mesh,
    scratch_types=(
        [pltpu.VMEM((_NCHUNK, _CHUNK), jnp.int32)]
        + [pltpu.VMEM((_BIG, _EMBED_DIM), jnp.float32)] * 2
        + [pltpu.SemaphoreType.DMA] * 4
    ),
)


def kernel(indices, table):
    flat_idx = indices.astype(jnp.int32).reshape(_NW, _NCHUNK, _CHUNK)
    out = _gather_call(flat_idx, table)
    return out.reshape(_BATCH, _HIST_LEN, _EMBED_DIM)


# speedup vs baseline: 1.0056x; 1.0056x over previous
import functools

import jax
import jax.numpy as jnp
from jax import lax
from jax.experimental import pallas as pl
from jax.experimental.pallas import tpu as pltpu
from jax.experimental.pallas import tpu_sc as plsc

_VOCAB = 100000
_EMBED_DIM = 128
_BATCH = 4096
_HIST_LEN = 200

_NUM_CORES = 2
_NUM_SUBCORES = 16
_NW = _NUM_CORES * _NUM_SUBCORES
_B_TOTAL = _BATCH * _HIST_LEN
_B_PER_W = _B_TOTAL // _NW
_CHUNK = 128
_NCHUNK = _B_PER_W // _CHUNK


_NBUF = 6
_DEPTH = 3
assert (2 * _DEPTH) % _NBUF == 0


def _gather_body(idx_hbm, table_hbm, out_hbm, idx_v, *rest):
    bufs = rest[:_NBUF]
    gsem = rest[_NBUF:2 * _NBUF]
    wsem = rest[2 * _NBUF:]
    wid = lax.axis_index("s") * _NUM_CORES + lax.axis_index("c")
    pltpu.sync_copy(idx_hbm.at[wid], idx_v)
    base = wid * _B_PER_W

    def start_gather(j, k):
        pltpu.async_copy(table_hbm.at[idx_v.at[j]], bufs[k], gsem[k])

    def wait_gather(j, k):
        pltpu.make_async_copy(table_hbm.at[idx_v.at[j]], bufs[k],
                              gsem[k]).wait()

    def out_slice(j):
        return out_hbm.at[pl.ds(base + j * _CHUNK, _CHUNK)]

    def start_write(j, k):
        pltpu.async_copy(bufs[k], out_slice(j), wsem[k])

    def wait_write(j, k):
        pltpu.make_async_copy(bufs[k], out_slice(j), wsem[k]).wait()

    def step_full(j, bj):
        kr = (bj + _DEPTH) % _NBUF
        wait_write(j - _DEPTH, kr)
        start_gather(j + _DEPTH, kr)
        wait_gather(j, bj)
        start_write(j, bj)

    for j in range(_DEPTH):
        start_gather(j, j % _NBUF)
    for j in range(_DEPTH):
        start_gather(j + _DEPTH, (j + _DEPTH) % _NBUF)
        wait_gather(j, j % _NBUF)
        start_write(j, j % _NBUF)

    groups, rem = divmod(_NCHUNK - 2 * _DEPTH, _NBUF)

    def group(g, carry):
        j0 = _DEPTH + g * _NBUF
        for k in range(_NBUF):
            step_full(j0 + k, (_DEPTH + k) % _NBUF)
        return carry

    lax.fori_loop(0, groups, group, 0)
    for i in range(rem):
        step_full(_DEPTH + groups * _NBUF + i, (_DEPTH + i) % _NBUF)

    for j in range(_NCHUNK - _DEPTH, _NCHUNK):
        bj = j % _NBUF
        wait_write(j - _DEPTH, (bj + _DEPTH) % _NBUF)
        wait_gather(j, bj)
        start_write(j, bj)
    for j in range(_NCHUNK - _DEPTH, _NCHUNK):
        wait_write(j, j % _NBUF)


_mesh = plsc.VectorSubcoreMesh(core_axis_name="c", subcore_axis_name="s")

_gather_call = pl.kernel(
    _gather_body,
    out_type=jax.ShapeDtypeStruct((_B_TOTAL, _EMBED_DIM), jnp.float32),
    mesh=_mesh,
    scratch_types=(
        [pltpu.VMEM((_NCHUNK, _CHUNK), jnp.int32)]
        + [pltpu.VMEM((_CHUNK, _EMBED_DIM), jnp.float32)] * _NBUF
        + [pltpu.SemaphoreType.DMA] * (2 * _NBUF)
    ),
)


def kernel(indices, table):
    flat_idx = indices.astype(jnp.int32).reshape(_NW, _NCHUNK, _CHUNK)
    out = _gather_call(flat_idx, table)
    return out.reshape(_BATCH, _HIST_LEN, _EMBED_DIM)
